# baseline (speedup 1.0000x reference)
import functools
import math

import jax
import jax.numpy as jnp
from jax import lax
from jax.experimental import pallas as pl
from jax.experimental.pallas import tpu as pltpu
from jax.experimental.pallas import tpu_sc as plsc

N = 10000
F = 256
NQ = 8
Q = 32
E = 160000
NC = 2
NS = 16
NPH = 4
CH = 128
NBUF = 8
ET = CH
NT = 80
EPT = ET * NT
EPAD = EPT * NS
NROWS = 10112
RPT = NROWS // NS
SPT = N // NS


def _agg_body(with_counts, xs_hbm, sidx_hbm, didx_hbm, zq_hbm, z8_hbm,
              ones_hbm, sums_out, cnt_out, idxs_v, idxd_v, bufs,
              ones_v, table_sh, accum, cnt_sh, sgs, sss, scs):
    c = lax.axis_index("c")
    s = lax.axis_index("s")
    base = s * RPT

    pltpu.sync_copy(sidx_hbm.at[s], idxs_v)
    pltpu.sync_copy(didx_hbm.at[s], idxd_v)
    if with_counts:
        pltpu.sync_copy(ones_hbm, ones_v)

    for p in range(NPH):
        pltpu.sync_copy(xs_hbm.at[pl.ds((2 * p + c) * N + SPT * s, SPT)],
                        table_sh.at[pl.ds(SPT * s, SPT)])
        pltpu.sync_copy(zq_hbm, accum.at[pl.ds(base, RPT)])
        if with_counts and p == 0:
            @pl.when(c == 0)
            def _zero_cnt():
                pltpu.sync_copy(z8_hbm, cnt_sh.at[pl.ds(base, RPT)])

        plsc.subcore_barrier()

        count_this_phase = with_counts and p == 0

        def outer(i, carry):
            j0 = i * NBUF
            gd = [pltpu.async_copy(
                table_sh.at[idxs_v.at[j0 + b]], bufs[b],
                sgs[b]) for b in range(NBUF)]
            sd = []
            for b in range(NBUF):
                gd[b].wait()
                sd.append(pltpu.async_copy(
                    bufs[b], accum.at[idxd_v.at[j0 + b]],
                    sss[b], add=True))
                if count_this_phase:
                    @pl.when(c == 0)
                    def _cnt():
                        pltpu.sync_copy(
                            ones_v,
                            cnt_sh.at[idxd_v.at[j0 + b]],
                            add=True)
            for b in range(NBUF):
                sd[b].wait()
            return carry

        lax.fori_loop(0, NT // NBUF, outer, 0)

        plsc.subcore_barrier()

        pltpu.sync_copy(accum.at[pl.ds(base, RPT)],
                        sums_out.at[2 * p + c, pl.ds(base, RPT)])
        if count_this_phase:
            @pl.when(c == 0)
            def _wb_cnt():
                pltpu.sync_copy(cnt_sh.at[pl.ds(base, RPT)],
                                cnt_out.at[pl.ds(base, RPT)])
        if p + 1 < NPH:
            plsc.subcore_barrier()


def _make_agg(with_counts):
    mesh = plsc.VectorSubcoreMesh(core_axis_name="c", subcore_axis_name="s",
                                  num_cores=NC, num_subcores=NS)
    cnt_rows = NROWS if with_counts else 8
    out_type = (jax.ShapeDtypeStruct((NQ, NROWS, Q), jnp.float32),
                jax.ShapeDtypeStruct((cnt_rows, 8), jnp.float32))
    scratch = [
        pltpu.VMEM((NT, ET), jnp.int32),
        pltpu.VMEM((NT, ET), jnp.int32),
        [pltpu.VMEM((ET, Q), jnp.float32) for _ in range(NBUF)],
        pltpu.VMEM((ET, 8) if with_counts else (8, 8), jnp.float32),
        pltpu.VMEM_SHARED((N, Q), jnp.float32),
        pltpu.VMEM_SHARED((NROWS, Q), jnp.float32),
        pltpu.VMEM_SHARED((cnt_rows, 8), jnp.float32),
        [pltpu.SemaphoreType.DMA for _ in range(NBUF)],
        [pltpu.SemaphoreType.DMA for _ in range(NBUF)],
        [pltpu.SemaphoreType.DMA for _ in range(2)],
    ]
    return pl.kernel(functools.partial(_agg_body, with_counts),
                     out_type=out_type, mesh=mesh, scratch_types=scratch,
                     compiler_params=pltpu.CompilerParams(
                         use_tc_tiling_on_sc=False),
                     name="sage_agg_cnt" if with_counts else "sage_agg")


_agg_with_counts = _make_agg(True)
_agg_plain = _make_agg(False)


def _dense_body(apply_gelu, slices_out, sums_ref, cnt_ref, x_ref,
                wl_ref, b_ref, wr_ref, out_ref):
    ssum = jnp.concatenate([sums_ref[q] for q in range(NQ)], axis=-1)
    cnt = cnt_ref[:, 0:1]
    recip = jnp.where(cnt > 0.0, 1.0 / jnp.maximum(cnt, 1.0), 0.0)
    mean = ssum * recip
    xin = jnp.concatenate([x_ref[q] for q in range(NQ)], axis=-1)
    acc = (jnp.dot(mean, wl_ref[...], preferred_element_type=jnp.float32)
           + b_ref[0:1, :]
           + jnp.dot(xin, wr_ref[...], preferred_element_type=jnp.float32))
    if apply_gelu:
        acc = 0.5 * acc * (1.0 + lax.erf(acc * (1.0 / math.sqrt(2.0))))
    if slices_out:
        for q in range(NQ):
            out_ref[q] = acc[:, q * Q:(q + 1) * Q]
    else:
        out_ref[...] = acc


def _dense(sums, cnt, x_slices, wl_t, b_pad, wr_t, apply_gelu, slices_out):
    R = 1000
    grid = (N // R,)
    in_specs = [
        pl.BlockSpec((NQ, R, Q), lambda i: (0, i, 0)),
        pl.BlockSpec((R, 8), lambda i: (i, 0)),
        pl.BlockSpec((NQ, R, Q), lambda i: (0, i, 0)),
        pl.BlockSpec((F, F), lambda i: (0, 0)),
        pl.BlockSpec((8, F), lambda i: (0, 0)),
        pl.BlockSpec((F, F), lambda i: (0, 0)),
    ]
    if slices_out:
        out_shape = jax.ShapeDtypeStruct((NQ, N, Q), jnp.float32)
        out_spec = pl.BlockSpec((NQ, R, Q), lambda i: (0, i, 0))
    else:
        out_shape = jax.ShapeDtypeStruct((N, F), jnp.float32)
        out_spec = pl.BlockSpec((R, F), lambda i: (i, 0))
    return pl.pallas_call(
        functools.partial(_dense_body, apply_gelu, slices_out),
        grid=grid, in_specs=in_specs, out_specs=out_spec,
        out_shape=out_shape,
    )(sums, cnt, x_slices, wl_t, b_pad, wr_t)


def kernel(x, edge_index, W_l0, b_l0, W_r0, W_l1, b_l1, W_r1):
    src = edge_index[0]
    dst = edge_index[1]
    pad = EPAD - E
    src_p = jnp.concatenate([src, jnp.zeros((pad,), jnp.int32)])
    dst_p = jnp.concatenate([dst, jnp.full((pad,), N, jnp.int32)])
    sidx = src_p.reshape(NS, NT, ET)
    didx = dst_p.reshape(NS, NT, ET)

    zq = jnp.zeros((RPT, Q), jnp.float32)
    z8 = jnp.zeros((RPT, 8), jnp.float32)
    ones8 = jnp.ones((ET, 8), jnp.float32)

    x_slices = x.reshape(N, NQ, Q).transpose(1, 0, 2)
    xs = x_slices.reshape(NQ * N, Q)

    sums1, cnt = _agg_with_counts(xs, sidx, didx, zq, z8, ones8)
    h_slices = _dense(sums1, cnt, x_slices, W_l0.T,
                      jnp.broadcast_to(b_l0[None, :], (8, F)), W_r0.T,
                      apply_gelu=True, slices_out=True)
    hs = h_slices.reshape(NQ * N, Q)
    sums2, _ = _agg_plain(hs, sidx, didx, zq, z8, ones8)
    out = _dense(sums2, cnt, h_slices, W_l1.T,
                 jnp.broadcast_to(b_l1[None, :], (8, F)), W_r1.T,
                 apply_gelu=False, slices_out=False)
    return out


# speedup vs baseline: 1.0012x; 1.0012x over previous
import functools
import math

import jax
import jax.numpy as jnp
from jax import lax
from jax.experimental import pallas as pl
from jax.experimental.pallas import tpu as pltpu
from jax.experimental.pallas import tpu_sc as plsc

N = 10000
F = 256
NQ = 8
Q = 32
E = 160000
NC = 2
NS = 16
NPH = 4
CH = 256
NBUF = 5
ET = CH
NT = 40
EPT = ET * NT
EPAD = EPT * NS
NROWS = 10112
RPT = NROWS // NS
SPT = N // NS


def _agg_body(with_counts, xs_hbm, sidx_hbm, didx_hbm, zq_hbm, z8_hbm,
              ones_hbm, sums_out, cnt_out, idxs_v, idxd_v, bufs,
              ones_v, table_sh, accum, cnt_sh, sgs, sss, scs):
    c = lax.axis_index("c")
    s = lax.axis_index("s")
    base = s * RPT

    pltpu.sync_copy(sidx_hbm.at[s], idxs_v)
    pltpu.sync_copy(didx_hbm.at[s], idxd_v)
    if with_counts:
        pltpu.sync_copy(ones_hbm, ones_v)

    for p in range(NPH):
        pltpu.sync_copy(xs_hbm.at[pl.ds((2 * p + c) * N + SPT * s, SPT)],
                        table_sh.at[pl.ds(SPT * s, SPT)])
        pltpu.sync_copy(zq_hbm, accum.at[pl.ds(base, RPT)])
        if with_counts and p == 0:
            @pl.when(c == 0)
            def _zero_cnt():
                pltpu.sync_copy(z8_hbm, cnt_sh.at[pl.ds(base, RPT)])

        plsc.subcore_barrier()

        count_this_phase = with_counts and p == 0

        def outer(i, carry):
            j0 = i * NBUF
            gd = [pltpu.async_copy(
                table_sh.at[idxs_v.at[j0 + b]], bufs[b],
                sgs[b]) for b in range(NBUF)]
            sd = []
            for b in range(NBUF):
                gd[b].wait()
                sd.append(pltpu.async_copy(
                    bufs[b], accum.at[idxd_v.at[j0 + b]],
                    sss[b], add=True))
                if count_this_phase:
                    @pl.when(c == 0)
                    def _cnt():
                        pltpu.sync_copy(
                            ones_v,
                            cnt_sh.at[idxd_v.at[j0 + b]],
                            add=True)
            for b in range(NBUF):
                sd[b].wait()
            return carry

        lax.fori_loop(0, NT // NBUF, outer, 0)

        plsc.subcore_barrier()

        pltpu.sync_copy(accum.at[pl.ds(base, RPT)],
                        sums_out.at[2 * p + c, pl.ds(base, RPT)])
        if count_this_phase:
            @pl.when(c == 0)
            def _wb_cnt():
                pltpu.sync_copy(cnt_sh.at[pl.ds(base, RPT)],
                                cnt_out.at[pl.ds(base, RPT)])
        if p + 1 < NPH:
            plsc.subcore_barrier()


def _make_agg(with_counts):
    mesh = plsc.VectorSubcoreMesh(core_axis_name="c", subcore_axis_name="s",
                                  num_cores=NC, num_subcores=NS)
    cnt_rows = NROWS if with_counts else 8
    out_type = (jax.ShapeDtypeStruct((NQ, NROWS, Q), jnp.float32),
                jax.ShapeDtypeStruct((cnt_rows, 8), jnp.float32))
    scratch = [
        pltpu.VMEM((NT, ET), jnp.int32),
        pltpu.VMEM((NT, ET), jnp.int32),
        [pltpu.VMEM((ET, Q), jnp.float32) for _ in range(NBUF)],
        pltpu.VMEM((ET, 8) if with_counts else (8, 8), jnp.float32),
        pltpu.VMEM_SHARED((N, Q), jnp.float32),
        pltpu.VMEM_SHARED((NROWS, Q), jnp.float32),
        pltpu.VMEM_SHARED((cnt_rows, 8), jnp.float32),
        [pltpu.SemaphoreType.DMA for _ in range(NBUF)],
        [pltpu.SemaphoreType.DMA for _ in range(NBUF)],
        [pltpu.SemaphoreType.DMA for _ in range(2)],
    ]
    return pl.kernel(functools.partial(_agg_body, with_counts),
                     out_type=out_type, mesh=mesh, scratch_types=scratch,
                     compiler_params=pltpu.CompilerParams(
                         use_tc_tiling_on_sc=False),
                     name="sage_agg_cnt" if with_counts else "sage_agg")


_agg_with_counts = _make_agg(True)
_agg_plain = _make_agg(False)


def _dense_body(apply_gelu, slices_out, sums_ref, cnt_ref, x_ref,
                wl_ref, b_ref, wr_ref, out_ref):
    ssum = jnp.concatenate([sums_ref[q] for q in range(NQ)], axis=-1)
    cnt = cnt_ref[:, 0:1]
    recip = jnp.where(cnt > 0.0, 1.0 / jnp.maximum(cnt, 1.0), 0.0)
    mean = ssum * recip
    xin = jnp.concatenate([x_ref[q] for q in range(NQ)], axis=-1)
    acc = (jnp.dot(mean, wl_ref[...], preferred_element_type=jnp.float32)
           + b_ref[0:1, :]
           + jnp.dot(xin, wr_ref[...], preferred_element_type=jnp.float32))
    if apply_gelu:
        acc = 0.5 * acc * (1.0 + lax.erf(acc * (1.0 / math.sqrt(2.0))))
    if slices_out:
        for q in range(NQ):
            out_ref[q] = acc[:, q * Q:(q + 1) * Q]
    else:
        out_ref[...] = acc


def _dense(sums, cnt, x_slices, wl_t, b_pad, wr_t, apply_gelu, slices_out):
    R = 1000
    grid = (N // R,)
    in_specs = [
        pl.BlockSpec((NQ, R, Q), lambda i: (0, i, 0)),
        pl.BlockSpec((R, 8), lambda i: (i, 0)),
        pl.BlockSpec((NQ, R, Q), lambda i: (0, i, 0)),
        pl.BlockSpec((F, F), lambda i: (0, 0)),
        pl.BlockSpec((8, F), lambda i: (0, 0)),
        pl.BlockSpec((F, F), lambda i: (0, 0)),
    ]
    if slices_out:
        out_shape = jax.ShapeDtypeStruct((NQ, N, Q), jnp.float32)
        out_spec = pl.BlockSpec((NQ, R, Q), lambda i: (0, i, 0))
    else:
        out_shape = jax.ShapeDtypeStruct((N, F), jnp.float32)
        out_spec = pl.BlockSpec((R, F), lambda i: (i, 0))
    return pl.pallas_call(
        functools.partial(_dense_body, apply_gelu, slices_out),
        grid=grid, in_specs=in_specs, out_specs=out_spec,
        out_shape=out_shape,
    )(sums, cnt, x_slices, wl_t, b_pad, wr_t)


def kernel(x, edge_index, W_l0, b_l0, W_r0, W_l1, b_l1, W_r1):
    src = edge_index[0]
    dst = edge_index[1]
    pad = EPAD - E
    src_p = jnp.concatenate([src, jnp.zeros((pad,), jnp.int32)])
    dst_p = jnp.concatenate([dst, jnp.full((pad,), N, jnp.int32)])
    sidx = src_p.reshape(NS, NT, ET)
    didx = dst_p.reshape(NS, NT, ET)

    zq = jnp.zeros((RPT, Q), jnp.float32)
    z8 = jnp.zeros((RPT, 8), jnp.float32)
    ones8 = jnp.ones((ET, 8), jnp.float32)

    x_slices = x.reshape(N, NQ, Q).transpose(1, 0, 2)
    xs = x_slices.reshape(NQ * N, Q)

    sums1, cnt = _agg_with_counts(xs, sidx, didx, zq, z8, ones8)
    h_slices = _dense(sums1, cnt, x_slices, W_l0.T,
                      jnp.broadcast_to(b_l0[None, :], (8, F)), W_r0.T,
                      apply_gelu=True, slices_out=True)
    hs = h_slices.reshape(NQ * N, Q)
    sums2, _ = _agg_plain(hs, sidx, didx, zq, z8, ones8)
    out = _dense(sums2, cnt, h_slices, W_l1.T,
                 jnp.broadcast_to(b_l1[None, :], (8, F)), W_r1.T,
                 apply_gelu=False, slices_out=False)
    return out


# speedup vs baseline: 1.0032x; 1.0020x over previous
import functools
import math

import jax
import jax.numpy as jnp
from jax import lax
from jax.experimental import pallas as pl
from jax.experimental.pallas import tpu as pltpu
from jax.experimental.pallas import tpu_sc as plsc

N = 10000
F = 256
NQ = 8
Q = 32
E = 160000
NC = 2
NS = 16
NPH = 4
CH = 256
NBUF = 5
ET = CH
NT = 40
EPT = ET * NT
EPAD = EPT * NS
NROWS = 10112
RPT = NROWS // NS
SPT = N // NS


def _agg_body(with_counts, xs_hbm, sidx_hbm, didx_hbm, zq_hbm, z8_hbm,
              ones_hbm, sums_out, cnt_out, idxs_v, idxd_v, bufs,
              ones_v, table_sh, accum, cnt_sh, sgs, sss, scs):
    c = lax.axis_index("c")
    s = lax.axis_index("s")
    base = s * RPT

    pltpu.sync_copy(sidx_hbm.at[s], idxs_v)
    pltpu.sync_copy(didx_hbm.at[s], idxd_v)
    if with_counts:
        pltpu.sync_copy(ones_hbm, ones_v)

    for p in range(NPH):
        pltpu.sync_copy(xs_hbm.at[pl.ds((2 * p + c) * N + SPT * s, SPT)],
                        table_sh.at[pl.ds(SPT * s, SPT)])
        pltpu.sync_copy(zq_hbm, accum.at[pl.ds(base, RPT)])
        if with_counts and p == 0:
            @pl.when(c == 0)
            def _zero_cnt():
                pltpu.sync_copy(z8_hbm, cnt_sh.at[pl.ds(base, RPT)])

        plsc.subcore_barrier()

        count_this_phase = with_counts and p == 0

        def outer(i, carry):
            j0 = i * NBUF
            gd = [pltpu.async_copy(
                table_sh.at[idxs_v.at[j0 + b]], bufs[b],
                sgs[b]) for b in range(NBUF)]
            sd = []
            for b in range(NBUF):
                gd[b].wait()
                sd.append(pltpu.async_copy(
                    bufs[b], accum.at[idxd_v.at[j0 + b]],
                    sss[b], add=True))
            if count_this_phase:
                @pl.when(c == 0)
                def _cnt():
                    cd = [pltpu.async_copy(
                        ones_v, cnt_sh.at[idxd_v.at[j0 + b]],
                        scs[b % 2], add=True) for b in range(NBUF)]
                    for d in cd:
                        d.wait()
            for b in range(NBUF):
                sd[b].wait()
            return carry

        lax.fori_loop(0, NT // NBUF, outer, 0)

        plsc.subcore_barrier()

        pltpu.sync_copy(accum.at[pl.ds(base, RPT)],
                        sums_out.at[2 * p + c, pl.ds(base, RPT)])
        if count_this_phase:
            @pl.when(c == 0)
            def _wb_cnt():
                pltpu.sync_copy(cnt_sh.at[pl.ds(base, RPT)],
                                cnt_out.at[pl.ds(base, RPT)])
        if p + 1 < NPH:
            plsc.subcore_barrier()


def _make_agg(with_counts):
    mesh = plsc.VectorSubcoreMesh(core_axis_name="c", subcore_axis_name="s",
                                  num_cores=NC, num_subcores=NS)
    cnt_rows = NROWS if with_counts else 8
    out_type = (jax.ShapeDtypeStruct((NQ, NROWS, Q), jnp.float32),
                jax.ShapeDtypeStruct((cnt_rows, 8), jnp.float32))
    scratch = [
        pltpu.VMEM((NT, ET), jnp.int32),
        pltpu.VMEM((NT, ET), jnp.int32),
        [pltpu.VMEM((ET, Q), jnp.float32) for _ in range(NBUF)],
        pltpu.VMEM((ET, 8) if with_counts else (8, 8), jnp.float32),
        pltpu.VMEM_SHARED((N, Q), jnp.float32),
        pltpu.VMEM_SHARED((NROWS, Q), jnp.float32),
        pltpu.VMEM_SHARED((cnt_rows, 8), jnp.float32),
        [pltpu.SemaphoreType.DMA for _ in range(NBUF)],
        [pltpu.SemaphoreType.DMA for _ in range(NBUF)],
        [pltpu.SemaphoreType.DMA for _ in range(2)],
    ]
    return pl.kernel(functools.partial(_agg_body, with_counts),
                     out_type=out_type, mesh=mesh, scratch_types=scratch,
                     compiler_params=pltpu.CompilerParams(
                         use_tc_tiling_on_sc=False),
                     name="sage_agg_cnt" if with_counts else "sage_agg")


_agg_with_counts = _make_agg(True)
_agg_plain = _make_agg(False)


def _dense_body(apply_gelu, slices_out, sums_ref, cnt_ref, x_ref,
                wl_ref, b_ref, wr_ref, out_ref):
    ssum = jnp.concatenate([sums_ref[q] for q in range(NQ)], axis=-1)
    cnt = cnt_ref[:, 0:1]
    recip = jnp.where(cnt > 0.0, 1.0 / jnp.maximum(cnt, 1.0), 0.0)
    mean = ssum * recip
    xin = jnp.concatenate([x_ref[q] for q in range(NQ)], axis=-1)
    acc = (jnp.dot(mean, wl_ref[...], preferred_element_type=jnp.float32)
           + b_ref[0:1, :]
           + jnp.dot(xin, wr_ref[...], preferred_element_type=jnp.float32))
    if apply_gelu:
        acc = 0.5 * acc * (1.0 + lax.erf(acc * (1.0 / math.sqrt(2.0))))
    if slices_out:
        for q in range(NQ):
            out_ref[q] = acc[:, q * Q:(q + 1) * Q]
    else:
        out_ref[...] = acc


def _dense(sums, cnt, x_slices, wl_t, b_pad, wr_t, apply_gelu, slices_out):
    R = 1000
    grid = (N // R,)
    in_specs = [
        pl.BlockSpec((NQ, R, Q), lambda i: (0, i, 0)),
        pl.BlockSpec((R, 8), lambda i: (i, 0)),
        pl.BlockSpec((NQ, R, Q), lambda i: (0, i, 0)),
        pl.BlockSpec((F, F), lambda i: (0, 0)),
        pl.BlockSpec((8, F), lambda i: (0, 0)),
        pl.BlockSpec((F, F), lambda i: (0, 0)),
    ]
    if slices_out:
        out_shape = jax.ShapeDtypeStruct((NQ, N, Q), jnp.float32)
        out_spec = pl.BlockSpec((NQ, R, Q), lambda i: (0, i, 0))
    else:
        out_shape = jax.ShapeDtypeStruct((N, F), jnp.float32)
        out_spec = pl.BlockSpec((R, F), lambda i: (i, 0))
    return pl.pallas_call(
        functools.partial(_dense_body, apply_gelu, slices_out),
        grid=grid, in_specs=in_specs, out_specs=out_spec,
        out_shape=out_shape,
    )(sums, cnt, x_slices, wl_t, b_pad, wr_t)


def kernel(x, edge_index, W_l0, b_l0, W_r0, W_l1, b_l1, W_r1):
    src = edge_index[0]
    dst = edge_index[1]
    pad = EPAD - E
    src_p = jnp.concatenate([src, jnp.zeros((pad,), jnp.int32)])
    dst_p = jnp.concatenate([dst, jnp.full((pad,), N, jnp.int32)])
    sidx = src_p.reshape(NS, NT, ET)
    didx = dst_p.reshape(NS, NT, ET)

    zq = jnp.zeros((RPT, Q), jnp.float32)
    z8 = jnp.zeros((RPT, 8), jnp.float32)
    ones8 = jnp.ones((ET, 8), jnp.float32)

    x_slices = x.reshape(N, NQ, Q).transpose(1, 0, 2)
    xs = x_slices.reshape(NQ * N, Q)

    sums1, cnt = _agg_with_counts(xs, sidx, didx, zq, z8, ones8)
    h_slices = _dense(sums1, cnt, x_slices, W_l0.T,
                      jnp.broadcast_to(b_l0[None, :], (8, F)), W_r0.T,
                      apply_gelu=True, slices_out=True)
    hs = h_slices.reshape(NQ * N, Q)
    sums2, _ = _agg_plain(hs, sidx, didx, zq, z8, ones8)
    out = _dense(sums2, cnt, h_slices, W_l1.T,
                 jnp.broadcast_to(b_l1[None, :], (8, F)), W_r1.T,
                 apply_gelu=False, slices_out=False)
    return out
